# Initial kernel scaffold; baseline (speedup 1.0000x reference)
import jax
import jax.numpy as jnp
from jax.experimental import pallas as pl


def kernel(logits, labels):
    raise NotImplementedError("write your pallas kernel here")


# speedup vs baseline: 7.7774x; 7.7774x over previous
import functools

import jax
import jax.numpy as jnp
import numpy as np
from jax import lax
from jax.experimental import pallas as pl
from jax.experimental.pallas import tpu as pltpu
from jax.experimental.pallas import tpu_sc as plsc

_THRESH = float(np.log(1.0 / 0.7))
_MIN_KEPT = 131072

_B, _C, _H, _W = 8, 19, 512, 512
_P = _H * _W
_ROWS = 256
_NBLK = (_P // 128) // _ROWS

_N = _B * _P

_NC = 2
_NS = 16
_NW = _NC * _NS
_CHUNK = _N // _NW


def _ce_body(lg_ref, lb_ref, loss_ref, s_ref, c_ref):
    x = lg_ref[0]
    lab = lb_ref[0]
    m = jnp.max(x, axis=0)
    e = jnp.exp(x - m[None])
    s = jnp.sum(e, axis=0)
    lse = m + jnp.log(s)
    cls = lax.broadcasted_iota(jnp.int32, (_C, _ROWS, 128), 0)
    picked = jnp.sum(jnp.where(cls == lab[None], x, 0.0), axis=0)
    loss = lse - picked
    loss_ref[0] = loss
    msk = loss > _THRESH
    ls = jnp.where(msk, loss, 0.0).reshape(_ROWS // 8, 8, 128)
    lc = msk.astype(jnp.float32).reshape(_ROWS // 8, 8, 128)

    @pl.when(jnp.logical_and(pl.program_id(0) == 0, pl.program_id(1) == 0))
    def _():
        s_ref[...] = jnp.zeros_like(s_ref)
        c_ref[...] = jnp.zeros_like(c_ref)

    s_ref[...] += jnp.sum(ls, axis=0)
    c_ref[...] += jnp.sum(lc, axis=0)


_ce_call = pl.pallas_call(
    _ce_body,
    grid=(_B, _NBLK),
    in_specs=[
        pl.BlockSpec((1, _C, _ROWS, 128), lambda i, j: (i, 0, j, 0)),
        pl.BlockSpec((1, _ROWS, 128), lambda i, j: (i, j, 0)),
    ],
    out_specs=[
        pl.BlockSpec((1, _ROWS, 128), lambda i, j: (i, j, 0)),
        pl.BlockSpec((8, 128), lambda i, j: (0, 0)),
        pl.BlockSpec((8, 128), lambda i, j: (0, 0)),
    ],
    out_shape=[
        jax.ShapeDtypeStruct((_B, _P // 128, 128), jnp.float32),
        jax.ShapeDtypeStruct((8, 128), jnp.float32),
        jax.ShapeDtypeStruct((8, 128), jnp.float32),
    ],
)


@functools.lru_cache(maxsize=None)
def _make_hist_kernel(shift, nbins):
    mesh = plsc.VectorSubcoreMesh(
        core_axis_name="c", subcore_axis_name="s", num_cores=_NC
    )

    @functools.partial(
        pl.kernel,
        mesh=mesh,
        out_type=(
            jax.ShapeDtypeStruct((_NW, nbins), jnp.int32),
            jax.ShapeDtypeStruct((_NW, nbins), jnp.float32),
        ),
        scratch_types=[
            pltpu.VMEM((_CHUNK,), jnp.float32),
            pltpu.VMEM((2, 16), jnp.int32),
            pltpu.VMEM((nbins,), jnp.int32),
            pltpu.VMEM((nbins,), jnp.float32),
        ],
        compiler_params=pltpu.CompilerParams(needs_layout_passes=False),
    )
    def hist(loss_hbm, state_hbm, cnt_out, sum_out, loss_v, state_v, cnt_v, sum_v):
        wid = lax.axis_index("s") * _NC + lax.axis_index("c")
        base = pl.multiple_of(wid * _CHUNK, 8)
        pltpu.sync_copy(loss_hbm.at[pl.ds(base, _CHUNK)], loss_v)
        pltpu.sync_copy(state_hbm, state_v)
        prefv = state_v[0, :]
        maskv = state_v[1, :]

        zi = jnp.zeros((16,), jnp.int32)
        zf = jnp.zeros((16,), jnp.float32)

        def zinit(j, carry):
            cnt_v[pl.ds(j * 16, 16)] = zi
            sum_v[pl.ds(j * 16, 16)] = zf
            return carry

        lax.fori_loop(0, nbins // 16, zinit, 0)

        ones = jnp.ones((16,), jnp.int32)
        lomask = jnp.int32(nbins - 1)
        sgn = jnp.int32(-2147483648)

        def body(i, carry):
            x = loss_v[pl.ds(i * 16, 16)]
            b = lax.bitcast_convert_type(x, jnp.int32)
            key = b ^ ((b >> 31) | sgn)
            match = (key & maskv) == prefv
            digit = (key >> shift) & lomask
            plsc.addupdate_scatter(cnt_v, [digit], ones, mask=match)
            plsc.addupdate_scatter(sum_v, [digit], x, mask=match)
            return carry

        lax.fori_loop(0, _CHUNK // 16, body, 0)

        pltpu.sync_copy(cnt_v, cnt_out.at[wid])
        pltpu.sync_copy(sum_v, sum_out.at[wid])

    return hist


_ROUNDS = ((20, 12), (8, 12), (0, 8))


def kernel(logits, labels):
    lg = logits.reshape(_B, _C, _P // 128, 128)
    lb = labels.reshape(_B, _P // 128, 128)
    loss3, s_acc, c_acc = _ce_call(lg, lb)
    loss = loss3.reshape(_N)
    sum_t = jnp.sum(s_acc)
    cnt_t = jnp.sum(c_acc)

    k = _MIN_KEPT
    prefix = jnp.int32(0)
    maskbits = jnp.int32(0)
    k_rem = jnp.int32(k)
    cnt_gt = jnp.int32(0)
    sum_gt = jnp.float32(0.0)
    for shift, nbits in _ROUNDS:
        nbins = 1 << nbits
        state = jnp.stack(
            [jnp.broadcast_to(prefix, (16,)), jnp.broadcast_to(maskbits, (16,))]
        )
        cnts, sums = _make_hist_kernel(shift, nbins)(loss, state)
        cnt = jnp.sum(cnts, axis=0)
        sm = jnp.sum(sums, axis=0)
        rc = jnp.cumsum(cnt[::-1])[::-1]
        dstar = jnp.sum((rc >= k_rem).astype(jnp.int32)) - 1
        above = rc[dstar] - cnt[dstar]
        s_above = jnp.sum(jnp.where(jnp.arange(nbins) > dstar, sm, 0.0))
        cnt_gt = cnt_gt + above
        sum_gt = sum_gt + s_above
        k_rem = k_rem - above
        prefix = prefix | jnp.left_shift(dstar, shift)
        maskbits = maskbits | jnp.left_shift(jnp.int32(nbins - 1), shift)

    sgn = jnp.int32(-2147483648)
    bb = jnp.where(prefix < 0, prefix ^ sgn, ~prefix)
    t = lax.bitcast_convert_type(bb, jnp.float32)
    mean_topk = (sum_gt + (jnp.float32(k) - cnt_gt.astype(jnp.float32)) * t) / k
    mean_thresh = sum_t / jnp.maximum(cnt_t, 1.0)
    cond = cnt_t > jnp.float32(_MIN_KEPT)
    return jnp.where(cond, mean_thresh, mean_topk)
